# baseline (speedup 1.0000x reference)
import functools

import jax
import jax.numpy as jnp
from jax import lax
from jax.experimental import pallas as pl
from jax.experimental.pallas import tpu as pltpu
from jax.experimental.pallas import tpu_sc as plsc

_N_ROWS = 819200
_D = 64
_NW = 32
_CHUNK = 128
_CPW = _N_ROWS // (_NW * _CHUNK)

_mesh = plsc.VectorSubcoreMesh(core_axis_name="c", subcore_axis_name="s")


@functools.partial(
    pl.kernel,
    out_type=jax.ShapeDtypeStruct((_N_ROWS, _D), jnp.float32),
    mesh=_mesh,
    scratch_types=[
        pltpu.VMEM((_CPW, _CHUNK), jnp.int32),
        pltpu.VMEM((_CHUNK, _D), jnp.float32),
        pltpu.SemaphoreType.DMA,
    ],
    compiler_params=pltpu.CompilerParams(use_tc_tiling_on_sc=False),
)
def _gather(table_hbm, idx_hbm, out_hbm, idx_v, rows_v, sem):
    wid = lax.axis_index("s") * 2 + lax.axis_index("c")
    base_chunk = wid * _CPW
    pltpu.sync_copy(idx_hbm.at[pl.ds(base_chunk, _CPW)], idx_v)

    def body(c, carry):
        pltpu.async_copy(table_hbm.at[idx_v.at[c]], rows_v, sem).wait()
        pltpu.sync_copy(
            rows_v, out_hbm.at[pl.ds((base_chunk + c) * _CHUNK, _CHUNK)]
        )
        return carry

    lax.fori_loop(0, _CPW, body, 0)


def kernel(x, weight):
    idx = jnp.reshape(x, (_N_ROWS // _CHUNK, _CHUNK))
    return _gather(weight, idx)


# speedup vs baseline: 1.1111x; 1.1111x over previous
import functools

import jax
import jax.numpy as jnp
from jax import lax
from jax.experimental import pallas as pl
from jax.experimental.pallas import tpu as pltpu
from jax.experimental.pallas import tpu_sc as plsc

_N_ROWS = 819200
_D = 64
_NW = 32
_CHUNK = 128
_CPW = _N_ROWS // (_NW * _CHUNK)
_K = 4
_GROUPS = _CPW // _K
_PAIRS = _GROUPS // 2

_mesh = plsc.VectorSubcoreMesh(core_axis_name="c", subcore_axis_name="s")


@functools.partial(
    pl.kernel,
    out_type=jax.ShapeDtypeStruct((_N_ROWS, _D), jnp.float32),
    mesh=_mesh,
    scratch_types=[
        pltpu.VMEM((_CPW, _CHUNK), jnp.int32),
        pltpu.VMEM((_K, _CHUNK, _D), jnp.float32),
        pltpu.VMEM((_K, _CHUNK, _D), jnp.float32),
        pltpu.SemaphoreType.DMA,
        pltpu.SemaphoreType.DMA,
        pltpu.SemaphoreType.DMA,
        pltpu.SemaphoreType.DMA,
    ],
    compiler_params=pltpu.CompilerParams(use_tc_tiling_on_sc=False),
)
def _gather(table_hbm, idx_hbm, out_hbm, idx_v, bufs0, bufs1,
            gsem0, gsem1, wsem0, wsem1):
    wid = lax.axis_index("s") * 2 + lax.axis_index("c")
    base_chunk = wid * _CPW
    pltpu.sync_copy(idx_hbm.at[pl.ds(base_chunk, _CPW)], idx_v)

    def out_slice(c):
        return out_hbm.at[pl.ds((base_chunk + c) * _CHUNK, _CHUNK)]

    def fire_gathers(g, bufs, sem):
        for b in range(_K):
            pltpu.async_copy(table_hbm.at[idx_v.at[g * _K + b]],
                             bufs.at[b], sem)

    def drain_gathers(g, bufs, sem):
        for b in range(_K):
            pltpu.make_async_copy(table_hbm.at[idx_v.at[g * _K + b]],
                                  bufs.at[b], sem).wait()

    def fire_writebacks(g, bufs, sem):
        for b in range(_K):
            pltpu.async_copy(bufs.at[b], out_slice(g * _K + b), sem)

    def drain_writebacks(g, bufs, sem):
        for b in range(_K):
            pltpu.make_async_copy(bufs.at[b], out_slice(g * _K + b),
                                  sem).wait()

    fire_gathers(0, bufs0, gsem0)

    def pair(p, carry):
        g0 = 2 * p
        g1 = g0 + 1
        drain_gathers(g0, bufs0, gsem0)

        @pl.when(p > 0)
        def _():
            drain_writebacks(g0 - 1, bufs1, wsem1)

        fire_gathers(g1, bufs1, gsem1)
        fire_writebacks(g0, bufs0, wsem0)
        drain_gathers(g1, bufs1, gsem1)
        drain_writebacks(g0, bufs0, wsem0)

        @pl.when(p < _PAIRS - 1)
        def _():
            fire_gathers(g1 + 1, bufs0, gsem0)

        fire_writebacks(g1, bufs1, wsem1)
        return carry

    lax.fori_loop(0, _PAIRS, pair, 0)
    drain_writebacks(_GROUPS - 1, bufs1, wsem1)


def kernel(x, weight):
    idx = jnp.reshape(x, (_N_ROWS // _CHUNK, _CHUNK))
    return _gather(weight, idx)


# speedup vs baseline: 1.1138x; 1.0024x over previous
import functools

import jax
import jax.numpy as jnp
from jax import lax
from jax.experimental import pallas as pl
from jax.experimental.pallas import tpu as pltpu
from jax.experimental.pallas import tpu_sc as plsc

_N_ROWS = 819200
_D = 64
_NW = 32
_CHUNK = 128
_KC = 4
_ROWS_PER_DMA = _KC * _CHUNK
_CPW = _N_ROWS // (_NW * _ROWS_PER_DMA)

_mesh = plsc.VectorSubcoreMesh(core_axis_name="c", subcore_axis_name="s")


@functools.partial(
    pl.kernel,
    out_type=jax.ShapeDtypeStruct((_N_ROWS, _D), jnp.float32),
    mesh=_mesh,
    scratch_types=[
        pltpu.VMEM((_CPW, _ROWS_PER_DMA), jnp.int32),
        pltpu.VMEM((_ROWS_PER_DMA, _D), jnp.float32),
        pltpu.VMEM((_ROWS_PER_DMA, _D), jnp.float32),
        pltpu.SemaphoreType.DMA,
        pltpu.SemaphoreType.DMA,
        pltpu.SemaphoreType.DMA,
        pltpu.SemaphoreType.DMA,
    ],
    compiler_params=pltpu.CompilerParams(use_tc_tiling_on_sc=False),
)
def _gather(table_hbm, idx_hbm, out_hbm, idx_v, buf0, buf1,
            gsem0, gsem1, wsem0, wsem1):
    wid = lax.axis_index("s") * 2 + lax.axis_index("c")
    base = wid * _CPW
    pltpu.sync_copy(idx_hbm.at[pl.ds(base, _CPW)], idx_v)

    def out_slice(t):
        return out_hbm.at[pl.ds((base + t) * _ROWS_PER_DMA, _ROWS_PER_DMA)]

    def gather(t, buf, sem):
        return pltpu.async_copy(table_hbm.at[idx_v.at[t]], buf, sem)

    def wb(t, buf, sem):
        return pltpu.async_copy(
            buf, out_slice(t), sem)

    gather(0, buf0, gsem0)

    def pair(p, carry):
        t0 = 2 * p
        t1 = t0 + 1
        pltpu.make_async_copy(table_hbm.at[idx_v.at[t0]], buf0, gsem0).wait()

        @pl.when(p > 0)
        def _():
            pltpu.make_async_copy(
                buf1, out_slice(t0 - 1),
                wsem1).wait()

        gather(t1, buf1, gsem1)
        wb(t0, buf0, wsem0)
        pltpu.make_async_copy(table_hbm.at[idx_v.at[t1]], buf1, gsem1).wait()
        pltpu.make_async_copy(
            buf0, out_slice(t0), wsem0).wait()

        @pl.when(p < _CPW // 2 - 1)
        def _():
            gather(t1 + 1, buf0, gsem0)

        wb(t1, buf1, wsem1)
        return carry

    lax.fori_loop(0, _CPW // 2, pair, 0)
    pltpu.make_async_copy(
        buf1, out_slice(_CPW - 1), wsem1).wait()


def kernel(x, weight):
    idx = jnp.reshape(x, (_N_ROWS // _ROWS_PER_DMA, _ROWS_PER_DMA))
    return _gather(weight, idx)


# speedup vs baseline: 1.1732x; 1.0533x over previous
import functools

import jax
import jax.numpy as jnp
from jax import lax
from jax.experimental import pallas as pl
from jax.experimental.pallas import tpu as pltpu
from jax.experimental.pallas import tpu_sc as plsc

_N_ROWS = 819200
_TABLE_ROWS = 1000000
_D = 64
_NW = 32
_CHUNK = 128
_KC = 4
_ROWS_PER_DMA = _KC * _CHUNK
_CPW = _N_ROWS // (_NW * _ROWS_PER_DMA)

_mesh = plsc.VectorSubcoreMesh(core_axis_name="c", subcore_axis_name="s")


@functools.partial(
    pl.kernel,
    out_type=jax.ShapeDtypeStruct((_N_ROWS, _D), jnp.float32),
    mesh=_mesh,
    scratch_types=[
        pltpu.VMEM((_CPW, _ROWS_PER_DMA), jnp.int32),
        pltpu.VMEM((_ROWS_PER_DMA, _D), jnp.float32),
        pltpu.VMEM((_ROWS_PER_DMA, _D), jnp.float32),
        pltpu.SemaphoreType.DMA,
        pltpu.SemaphoreType.DMA,
        pltpu.SemaphoreType.DMA,
        pltpu.SemaphoreType.DMA,
    ],
    compiler_params=pltpu.CompilerParams(use_tc_tiling_on_sc=False),
)
def _gather(table_hbm, idx_hbm, out_hbm, idx_v, buf0, buf1,
            gsem0, gsem1, wsem0, wsem1):
    wid = lax.axis_index("s") * 2 + lax.axis_index("c")
    base = wid * _CPW
    pltpu.sync_copy(idx_hbm.at[pl.ds(base, _CPW)], idx_v)

    def out_slice(t):
        return out_hbm.at[pl.ds((base + t) * _ROWS_PER_DMA, _ROWS_PER_DMA)]

    def gather(t, buf, sem):
        return pltpu.async_copy(table_hbm.at[idx_v.at[t]], buf, sem)

    def wb(t, buf, sem):
        return pltpu.async_copy(
            buf, out_slice(t), sem)

    gather(0, buf0, gsem0)

    def pair(p, carry):
        t0 = 2 * p
        t1 = t0 + 1
        pltpu.make_async_copy(table_hbm.at[idx_v.at[t0]], buf0, gsem0).wait()

        @pl.when(p > 0)
        def _():
            pltpu.make_async_copy(
                buf1, out_slice(t0 - 1),
                wsem1).wait()

        gather(t1, buf1, gsem1)
        wb(t0, buf0, wsem0)
        pltpu.make_async_copy(table_hbm.at[idx_v.at[t1]], buf1, gsem1).wait()
        pltpu.make_async_copy(
            buf0, out_slice(t0), wsem0).wait()

        @pl.when(p < _CPW // 2 - 1)
        def _():
            gather(t1 + 1, buf0, gsem0)

        wb(t1, buf1, wsem1)
        return carry

    lax.fori_loop(0, _CPW // 2, pair, 0)
    pltpu.make_async_copy(
        buf1, out_slice(_CPW - 1), wsem1).wait()


def kernel(x, weight):
    wt = jnp.pad(weight, ((0, 0), (0, 64))).reshape(2 * _TABLE_ROWS, _D)
    idx = jnp.reshape(x * 2, (_N_ROWS // _ROWS_PER_DMA, _ROWS_PER_DMA))
    return _gather(wt, idx)
